# Initial kernel scaffold; baseline (speedup 1.0000x reference)
import jax
import jax.numpy as jnp
from jax.experimental import pallas as pl


def kernel(features_3, features_4, features_5, features_6, neighbors_3, neighbors_4, neighbors_5, neighbors_6, W1, b1, W2, b2, W3, b3, W4, b4):
    raise NotImplementedError("write your pallas kernel here")


# speedup vs baseline: 41.9910x; 41.9910x over previous
import functools

import jax
import jax.numpy as jnp
from jax import lax
from jax.experimental import pallas as pl
from jax.experimental.pallas import tpu as pltpu
from jax.experimental.pallas import tpu_sc as plsc

NC = 2
NS = 16
NW = NC * NS


def _sc_gather(table, idx, chunk):
  B = idx.shape[0]
  assert B % NW == 0
  bw = B // NW
  assert bw % chunk == 0 and chunk % 8 == 0
  n_it = bw // chunk
  d1 = table.ndim == 1
  row_shape = (chunk,) if d1 else (chunk, table.shape[1])
  out_shape = (B,) if d1 else (B, table.shape[1])

  mesh = plsc.VectorSubcoreMesh(core_axis_name="c", subcore_axis_name="s")

  @functools.partial(
      pl.kernel,
      out_type=jax.ShapeDtypeStruct(out_shape, jnp.float32),
      mesh=mesh,
      compiler_params=pltpu.CompilerParams(use_tc_tiling_on_sc=False),
      scratch_types=[
          pltpu.VMEM((chunk,), jnp.int32),
          pltpu.VMEM(row_shape, jnp.float32),
          pltpu.SemaphoreType.DMA,
      ],
  )
  def k(table_hbm, idx_hbm, out_hbm, idx_v, rows_v, sem):
    wid = lax.axis_index("s") * NC + lax.axis_index("c")
    base = wid * bw

    def body(it, carry):
      off = base + it * chunk
      pltpu.sync_copy(idx_hbm.at[pl.ds(off, chunk)], idx_v)
      pltpu.async_copy(table_hbm.at[idx_v], rows_v, sem).wait()
      if d1:
        pltpu.sync_copy(rows_v, out_hbm.at[pl.ds(off, chunk)])
      else:
        pltpu.sync_copy(rows_v, out_hbm.at[pl.ds(off, chunk), :])
      return carry

    lax.fori_loop(0, n_it, body, 0)

  return k(table, idx)


def _tc_matmul_relu(a, w, b, block_rows=2048):
  N, K = a.shape
  O = w.shape[1]
  grid = (pl.cdiv(N, block_rows),)

  def body(a_ref, w_ref, b_ref, o_ref):
    acc = jnp.dot(a_ref[...], w_ref[...], preferred_element_type=jnp.float32)
    o_ref[...] = jnp.maximum(acc + b_ref[...], 0.0)

  return pl.pallas_call(
      body,
      grid=grid,
      in_specs=[
          pl.BlockSpec((block_rows, K), lambda i: (i, 0)),
          pl.BlockSpec((K, O), lambda i: (0, 0)),
          pl.BlockSpec((1, O), lambda i: (0, 0)),
      ],
      out_specs=pl.BlockSpec((block_rows, O), lambda i: (i, 0)),
      out_shape=jax.ShapeDtypeStruct((N, O), jnp.float32),
  )(a, w, b)


def kernel(features_3, features_4, features_5, features_6,
           neighbors_3, neighbors_4, neighbors_5, neighbors_6,
           W1, b1, W2, b2, W3, b3, W4, b4):
  sizes = [f.shape[0] for f in
           (features_3, features_4, features_5, features_6)]
  offs = [0, sizes[0], sizes[0] + sizes[1], sizes[0] + sizes[1] + sizes[2]]
  N = sum(sizes)

  x = jnp.concatenate(
      [features_3, features_4, features_5, features_6], axis=0)[:, 0]
  nidx = jnp.concatenate(
      [neighbors_3 + offs[0], neighbors_4 + offs[1],
       neighbors_5 + offs[2], neighbors_6 + offs[3]], axis=0)
  flat_idx = nidx.reshape(-1)

  def wt(W):
    O, C, D = W.shape
    return jnp.transpose(W, (2, 1, 0)).reshape(D * C, O)

  w1, w2, w3, w4 = wt(W1), wt(W2), wt(W3), wt(W4)
  bb1, bb2, bb3, bb4 = (b.reshape(1, -1) for b in (b1, b2, b3, b4))

  g1 = _sc_gather(x, flat_idx, 6480).reshape(N, 9)
  h1 = _tc_matmul_relu(g1, w1, bb1)

  g2 = _sc_gather(h1, flat_idx, 3120).reshape(N, 9 * 16)
  h2 = _tc_matmul_relu(g2, w2, bb2)

  g3 = _sc_gather(h2, flat_idx, 3120).reshape(N, 9 * 32)
  h3 = _tc_matmul_relu(g3, w3, bb3)

  g4 = _sc_gather(h3, flat_idx, 3120).reshape(N, 9 * 16)
  out = _tc_matmul_relu(g4, w4, bb4)

  s0, s1, s2, s3 = sizes
  return (out[:s0], out[s0:s0 + s1],
          out[s0 + s1:s0 + s1 + s2], out[s0 + s1 + s2:])


# speedup vs baseline: 54.0865x; 1.2880x over previous
import functools

import jax
import jax.numpy as jnp
from jax import lax
from jax.experimental import pallas as pl
from jax.experimental.pallas import tpu as pltpu
from jax.experimental.pallas import tpu_sc as plsc

NC = 2
NS = 16
NW = NC * NS

_SC_PARAMS = pltpu.CompilerParams(use_tc_tiling_on_sc=False,
                                  needs_layout_passes=False)


def _sc_gather(table, idx, chunk):
  B = idx.shape[0]
  assert B % NW == 0
  bw = B // NW
  assert bw % chunk == 0 and chunk % 8 == 0
  n_it = bw // chunk
  d1 = table.ndim == 1
  row_shape = (chunk,) if d1 else (chunk, table.shape[1])
  out_shape = (B,) if d1 else (B, table.shape[1])

  mesh = plsc.VectorSubcoreMesh(core_axis_name="c", subcore_axis_name="s")

  @functools.partial(
      pl.kernel,
      out_type=jax.ShapeDtypeStruct(out_shape, jnp.float32),
      mesh=mesh,
      compiler_params=_SC_PARAMS,
      scratch_types=[
          pltpu.VMEM((chunk,), jnp.int32),
          pltpu.VMEM(row_shape, jnp.float32),
          pltpu.SemaphoreType.DMA,
      ],
  )
  def k(table_hbm, idx_hbm, out_hbm, idx_v, rows_v, sem):
    wid = lax.axis_index("s") * NC + lax.axis_index("c")
    base = wid * bw

    def body(it, carry):
      off = base + it * chunk
      pltpu.sync_copy(idx_hbm.at[pl.ds(off, chunk)], idx_v)
      pltpu.async_copy(table_hbm.at[idx_v], rows_v, sem).wait()
      if d1:
        pltpu.sync_copy(rows_v, out_hbm.at[pl.ds(off, chunk)])
      else:
        pltpu.sync_copy(rows_v, out_hbm.at[pl.ds(off, chunk), :])
      return carry

    lax.fori_loop(0, n_it, body, 0)

  return k(table, idx)


def _sc_gather9sum16(z, fidx, bias, nch):
  NT = fidx.shape[0]
  N = NT // 9
  assert N % NW == 0
  nodes_w = N // NW
  assert nodes_w % nch == 0 and (9 * nch) % 8 == 0
  n_it = nodes_w // nch
  ich = 9 * nch

  mesh = plsc.VectorSubcoreMesh(core_axis_name="c", subcore_axis_name="s")

  @functools.partial(
      pl.kernel,
      out_type=jax.ShapeDtypeStruct((N, 16), jnp.float32),
      mesh=mesh,
      compiler_params=_SC_PARAMS,
      scratch_types=[
          pltpu.VMEM((ich,), jnp.int32),
          pltpu.VMEM((ich, 16), jnp.float32),
          pltpu.VMEM((nch, 16), jnp.float32),
          pltpu.VMEM((16,), jnp.float32),
          pltpu.SemaphoreType.DMA,
      ],
  )
  def k(z_hbm, idx_hbm, b_hbm, out_hbm, idx_v, rows_v, out_v, b_v, sem):
    wid = lax.axis_index("s") * NC + lax.axis_index("c")
    nbase = wid * nodes_w
    pltpu.sync_copy(b_hbm, b_v)

    def body(it, carry):
      noff = nbase + it * nch
      ioff = noff * 9
      pltpu.sync_copy(idx_hbm.at[pl.ds(ioff, ich)], idx_v)
      pltpu.async_copy(z_hbm.at[idx_v], rows_v, sem).wait()
      bias_vec = b_v[...]

      def node(i, c):
        acc = rows_v[i * 9]
        for d in range(1, 9):
          acc = acc + rows_v[i * 9 + d]
        out_v[i] = jnp.maximum(acc + bias_vec, 0.0)
        return c

      lax.fori_loop(0, nch, node, 0)
      pltpu.sync_copy(out_v, out_hbm.at[pl.ds(noff, nch), :])
      return carry

    lax.fori_loop(0, n_it, body, 0)

  return k(z, fidx, bias)


def _sc_gather9sum1(z, fidx, bias, nch):
  NT = fidx.shape[0]
  N = NT // 9
  assert N % NW == 0
  nodes_w = N // NW
  assert nodes_w % nch == 0 and nch % 16 == 0
  n_it = nodes_w // nch
  ich = 9 * nch

  mesh = plsc.VectorSubcoreMesh(core_axis_name="c", subcore_axis_name="s")

  @functools.partial(
      pl.kernel,
      out_type=jax.ShapeDtypeStruct((N,), jnp.float32),
      mesh=mesh,
      compiler_params=_SC_PARAMS,
      scratch_types=[
          pltpu.VMEM((ich,), jnp.int32),
          pltpu.VMEM((ich,), jnp.float32),
          pltpu.VMEM((nch,), jnp.float32),
          pltpu.VMEM((16,), jnp.float32),
          pltpu.SemaphoreType.DMA,
      ],
  )
  def k(z_hbm, idx_hbm, b_hbm, out_hbm, idx_v, rows_v, out_v, b_v, sem):
    wid = lax.axis_index("s") * NC + lax.axis_index("c")
    nbase = wid * nodes_w
    pltpu.sync_copy(b_hbm, b_v)

    def body(it, carry):
      noff = nbase + it * nch
      ioff = noff * 9
      pltpu.sync_copy(idx_hbm.at[pl.ds(ioff, ich)], idx_v)
      pltpu.async_copy(z_hbm.at[idx_v], rows_v, sem).wait()
      bias_vec = b_v[...]
      lane9 = lax.iota(jnp.int32, 16) * 9

      def group(g, c):
        base = g * 144
        acc = bias_vec
        for d in range(9):
          acc = acc + plsc.load_gather(rows_v, [lane9 + (base + d)])
        out_v[pl.ds(g * 16, 16)] = jnp.maximum(acc, 0.0)
        return c

      lax.fori_loop(0, nch // 16, group, 0)
      pltpu.sync_copy(out_v, out_hbm.at[pl.ds(noff, nch)])
      return carry

    lax.fori_loop(0, n_it, body, 0)

  return k(z, fidx, bias)


def _tc_matmul(a, w, b=None, relu=False, w2=None, block_rows=2048):
  N, K = a.shape
  O = w.shape[1]
  O2 = w2.shape[1] if w2 is not None else O
  grid = (pl.cdiv(N, block_rows),)

  def body(*refs):
    if b is not None and w2 is not None:
      a_ref, w_ref, b_ref, w2_ref, o_ref = refs
    elif b is not None:
      a_ref, w_ref, b_ref, o_ref = refs
      w2_ref = None
    else:
      a_ref, w_ref, o_ref = refs
      b_ref = w2_ref = None
    h = jnp.dot(a_ref[...], w_ref[...], preferred_element_type=jnp.float32)
    if b_ref is not None:
      h = h + b_ref[...]
    if relu:
      h = jnp.maximum(h, 0.0)
    if w2_ref is not None:
      h = jnp.dot(h, w2_ref[...], preferred_element_type=jnp.float32)
    o_ref[...] = h

  in_specs = [pl.BlockSpec((block_rows, K), lambda i: (i, 0)),
              pl.BlockSpec((K, O), lambda i: (0, 0))]
  args = [a, w]
  if b is not None:
    in_specs.append(pl.BlockSpec((1, O), lambda i: (0, 0)))
    args.append(b)
  if w2 is not None:
    in_specs.append(pl.BlockSpec((O, O2), lambda i: (0, 0)))
    args.append(w2)

  return pl.pallas_call(
      body,
      grid=grid,
      in_specs=in_specs,
      out_specs=pl.BlockSpec((block_rows, O2), lambda i: (i, 0)),
      out_shape=jax.ShapeDtypeStruct((N, O2), jnp.float32),
  )(*args)


def kernel(features_3, features_4, features_5, features_6,
           neighbors_3, neighbors_4, neighbors_5, neighbors_6,
           W1, b1, W2, b2, W3, b3, W4, b4):
  sizes = [f.shape[0] for f in
           (features_3, features_4, features_5, features_6)]
  offs = [0, sizes[0], sizes[0] + sizes[1], sizes[0] + sizes[1] + sizes[2]]
  N = sum(sizes)

  x = jnp.concatenate(
      [features_3, features_4, features_5, features_6], axis=0)[:, 0]
  nidx = jnp.concatenate(
      [neighbors_3 + offs[0], neighbors_4 + offs[1],
       neighbors_5 + offs[2], neighbors_6 + offs[3]], axis=0)
  flat_idx = nidx.reshape(-1)
  fidx = (nidx * 9 + jnp.arange(9, dtype=jnp.int32)).reshape(-1)

  w1 = jnp.transpose(W1, (2, 1, 0)).reshape(9 * 1, 16)
  w2 = jnp.transpose(W2, (2, 1, 0)).reshape(9 * 16, 32)
  w3 = jnp.transpose(W3, (1, 2, 0)).reshape(32, 9 * 16)
  w4 = W4.reshape(16, 9)
  bb1 = b1.reshape(1, 16)
  bb2 = b2.reshape(1, 32)
  bb3 = jnp.broadcast_to(b3, (16,))
  bb4 = jnp.broadcast_to(b4, (16,))

  g1 = _sc_gather(x, flat_idx, 6480).reshape(N, 9)
  h1 = _tc_matmul(g1, w1, bb1, relu=True)

  g2 = _sc_gather(h1, flat_idx, 3120).reshape(N, 9 * 16)
  z3 = _tc_matmul(g2, w2, bb2, relu=True, w2=w3)

  h3 = _sc_gather9sum16(z3.reshape(N * 9, 16), fidx, bb3, 360)

  z4 = _tc_matmul(h3, w4)
  out = _sc_gather9sum1(z4.reshape(N * 9), fidx, bb4, 720)

  s0, s1, s2, s3 = sizes
  out = out.reshape(N, 1)
  return (out[:s0], out[s0:s0 + s1],
          out[s0 + s1:s0 + s1 + s2], out[s0 + s1 + s2:])


# speedup vs baseline: 54.1370x; 1.0009x over previous
import functools

import jax
import jax.numpy as jnp
from jax import lax
from jax.experimental import pallas as pl
from jax.experimental.pallas import tpu as pltpu
from jax.experimental.pallas import tpu_sc as plsc

NC = 2
NS = 16
NW = NC * NS

_SC_PARAMS = pltpu.CompilerParams(use_tc_tiling_on_sc=False,
                                  needs_layout_passes=False)


def _bf16_round(v):
  u = plsc.bitcast(v, jnp.int32)
  r = (u + jnp.int32(0x7FFF) + ((u >> 16) & 1)) & jnp.int32(-65536)
  return plsc.bitcast(r, jnp.float32)


def _sc_gather(table, idx, chunk):
  B = idx.shape[0]
  assert B % NW == 0
  bw = B // NW
  assert bw % chunk == 0 and chunk % 8 == 0
  n_it = bw // chunk
  d1 = table.ndim == 1
  row_shape = (chunk,) if d1 else (chunk, table.shape[1])
  out_shape = (B,) if d1 else (B, table.shape[1])

  mesh = plsc.VectorSubcoreMesh(core_axis_name="c", subcore_axis_name="s")

  @functools.partial(
      pl.kernel,
      out_type=jax.ShapeDtypeStruct(out_shape, jnp.float32),
      mesh=mesh,
      compiler_params=_SC_PARAMS,
      scratch_types=[
          pltpu.VMEM((chunk,), jnp.int32),
          pltpu.VMEM(row_shape, jnp.float32),
          pltpu.SemaphoreType.DMA,
      ],
  )
  def k(table_hbm, idx_hbm, out_hbm, idx_v, rows_v, sem):
    wid = lax.axis_index("s") * NC + lax.axis_index("c")
    base = wid * bw

    def body(it, carry):
      off = base + it * chunk
      pltpu.sync_copy(idx_hbm.at[pl.ds(off, chunk)], idx_v)
      pltpu.async_copy(table_hbm.at[idx_v], rows_v, sem).wait()
      if d1:
        pltpu.sync_copy(rows_v, out_hbm.at[pl.ds(off, chunk)])
      else:
        pltpu.sync_copy(rows_v, out_hbm.at[pl.ds(off, chunk), :])
      return carry

    lax.fori_loop(0, n_it, body, 0)

  return k(table, idx)


def _sc_gather9sum16(z, fidx, bias, nch):
  NT = fidx.shape[0]
  N = NT // 9
  assert N % NW == 0
  nodes_w = N // NW
  assert nodes_w % nch == 0 and (9 * nch) % 8 == 0
  n_it = nodes_w // nch
  ich = 9 * nch

  mesh = plsc.VectorSubcoreMesh(core_axis_name="c", subcore_axis_name="s")

  @functools.partial(
      pl.kernel,
      out_type=jax.ShapeDtypeStruct((N, 16), jnp.float32),
      mesh=mesh,
      compiler_params=_SC_PARAMS,
      scratch_types=[
          pltpu.VMEM((ich,), jnp.int32),
          pltpu.VMEM((ich, 16), jnp.float32),
          pltpu.VMEM((nch, 16), jnp.float32),
          pltpu.VMEM((16,), jnp.float32),
          pltpu.SemaphoreType.DMA,
      ],
  )
  def k(z_hbm, idx_hbm, b_hbm, out_hbm, idx_v, rows_v, out_v, b_v, sem):
    wid = lax.axis_index("s") * NC + lax.axis_index("c")
    nbase = wid * nodes_w
    pltpu.sync_copy(b_hbm, b_v)

    def body(it, carry):
      noff = nbase + it * nch
      ioff = noff * 9
      pltpu.sync_copy(idx_hbm.at[pl.ds(ioff, ich)], idx_v)
      pltpu.async_copy(z_hbm.at[idx_v], rows_v, sem).wait()
      bias_vec = b_v[...]

      def node(i, c):
        acc = rows_v[i * 9]
        for d in range(1, 9):
          acc = acc + rows_v[i * 9 + d]
        out_v[i] = jnp.maximum(acc + bias_vec, 0.0)
        return c

      lax.fori_loop(0, nch, node, 0)
      pltpu.sync_copy(out_v, out_hbm.at[pl.ds(noff, nch), :])
      return carry

    lax.fori_loop(0, n_it, body, 0)

  return k(z, fidx, bias)


def _sc_gather1mm(x, idx, w, b, nch):
  NT = idx.shape[0]
  N = NT // 9
  assert N % NW == 0
  nodes_w = N // NW
  assert nodes_w % nch == 0 and nch % 16 == 0
  n_it = nodes_w // nch
  ich = 9 * nch

  mesh = plsc.VectorSubcoreMesh(core_axis_name="c", subcore_axis_name="s")

  @functools.partial(
      pl.kernel,
      out_type=jax.ShapeDtypeStruct((N, 16), jnp.float32),
      mesh=mesh,
      compiler_params=_SC_PARAMS,
      scratch_types=[
          pltpu.VMEM((ich,), jnp.int32),
          pltpu.VMEM((ich,), jnp.float32),
          pltpu.VMEM((nch, 16), jnp.float32),
          pltpu.VMEM((9, 16, 16), jnp.float32),
          pltpu.VMEM((16, 16), jnp.float32),
          pltpu.SemaphoreType.DMA,
      ],
  )
  def k(x_hbm, idx_hbm, w_hbm, b_hbm, out_hbm, idx_v, rows_v, out_v, w_v,
        b_v, sem):
    wid = lax.axis_index("s") * NC + lax.axis_index("c")
    nbase = wid * nodes_w
    pltpu.sync_copy(w_hbm, w_v)
    pltpu.sync_copy(b_hbm, b_v)

    def body(it, carry):
      noff = nbase + it * nch
      ioff = noff * 9
      pltpu.sync_copy(idx_hbm.at[pl.ds(ioff, ich)], idx_v)
      pltpu.async_copy(x_hbm.at[idx_v], rows_v, sem).wait()
      lane = lax.iota(jnp.int32, 16)
      lane9 = lane * 9

      def group(g, c):
        gv = [plsc.load_gather(rows_v, [lane9 + (g * 144 + d)])
              for d in range(9)]
        rows = lane + g * 16
        for o in range(16):
          acc = b_v[o]
          for d in range(9):
            acc = acc + gv[d] * w_v[d, o]
          acc = jnp.maximum(acc, 0.0)
          plsc.store_scatter(
              out_v, [rows, jnp.broadcast_to(jnp.int32(o), (16,))], acc)
        return c

      lax.fori_loop(0, nch // 16, group, 0)
      pltpu.sync_copy(out_v, out_hbm.at[pl.ds(noff, nch), :])
      return carry

    lax.fori_loop(0, n_it, body, 0)

  return k(x, idx, w, b)


def _sc_gather9sum16mm(z, fidx, bias, wm, wh, nch):
  NT = fidx.shape[0]
  N = NT // 9
  assert N % NW == 0
  nodes_w = N // NW
  assert nodes_w % nch == 0 and nch % 16 == 0
  n_it = nodes_w // nch
  ich = 9 * nch

  mesh = plsc.VectorSubcoreMesh(core_axis_name="c", subcore_axis_name="s")

  @functools.partial(
      pl.kernel,
      out_type=jax.ShapeDtypeStruct((NT,), jnp.float32),
      mesh=mesh,
      compiler_params=_SC_PARAMS,
      scratch_types=[
          pltpu.VMEM((ich,), jnp.int32),
          pltpu.VMEM((ich, 16), jnp.float32),
          pltpu.VMEM((nch, 16), jnp.float32),
          pltpu.VMEM((ich,), jnp.float32),
          pltpu.VMEM((16, 9, 16), jnp.float32),
          pltpu.VMEM((16, 9, 16), jnp.float32),
          pltpu.VMEM((16,), jnp.float32),
          pltpu.SemaphoreType.DMA,
      ],
  )
  def k(z_hbm, idx_hbm, b_hbm, wm_hbm, wh_hbm, out_hbm, idx_v, rows_v, h_v,
        z4_v, wm_v, wh_v, b_v, sem):
    wid = lax.axis_index("s") * NC + lax.axis_index("c")
    nbase = wid * nodes_w
    pltpu.sync_copy(wm_hbm, wm_v)
    pltpu.sync_copy(wh_hbm, wh_v)
    pltpu.sync_copy(b_hbm, b_v)

    def body(it, carry):
      noff = nbase + it * nch
      ioff = noff * 9
      pltpu.sync_copy(idx_hbm.at[pl.ds(ioff, ich)], idx_v)
      pltpu.async_copy(z_hbm.at[idx_v], rows_v, sem).wait()
      bias_vec = b_v[...]

      def node(i, c):
        acc = rows_v[i * 9]
        for d in range(1, 9):
          acc = acc + rows_v[i * 9 + d]
        h_v[i] = jnp.maximum(acc + bias_vec, 0.0)
        return c

      lax.fori_loop(0, nch, node, 0)

      lane = lax.iota(jnp.int32, 16)
      lane9 = lane * 9

      def group(g, c):
        rows = lane + g * 16
        cols = [
            plsc.load_gather(
                h_v, [rows, jnp.broadcast_to(jnp.int32(cc), (16,))])
            for cc in range(16)
        ]
        for d in range(9):
          acc = cols[0] * wm_v[0, d]
          for cc in range(1, 16):
            acc = acc + cols[cc] * wm_v[cc, d]
          plsc.store_scatter(z4_v, [lane9 + (g * 144 + d)], acc)
        return c

      lax.fori_loop(0, nch // 16, group, 0)
      pltpu.sync_copy(z4_v, out_hbm.at[pl.ds(ioff, ich)])
      return carry

    lax.fori_loop(0, n_it, body, 0)

  return k(z, fidx, bias, wm, wh)


def _sc_gather9sum1(z, fidx, bias, nch):
  NT = fidx.shape[0]
  N = NT // 9
  assert N % NW == 0
  nodes_w = N // NW
  assert nodes_w % nch == 0 and nch % 16 == 0
  n_it = nodes_w // nch
  ich = 9 * nch

  mesh = plsc.VectorSubcoreMesh(core_axis_name="c", subcore_axis_name="s")

  @functools.partial(
      pl.kernel,
      out_type=jax.ShapeDtypeStruct((N,), jnp.float32),
      mesh=mesh,
      compiler_params=_SC_PARAMS,
      scratch_types=[
          pltpu.VMEM((ich,), jnp.int32),
          pltpu.VMEM((ich,), jnp.float32),
          pltpu.VMEM((nch,), jnp.float32),
          pltpu.VMEM((16,), jnp.float32),
          pltpu.SemaphoreType.DMA,
      ],
  )
  def k(z_hbm, idx_hbm, b_hbm, out_hbm, idx_v, rows_v, out_v, b_v, sem):
    wid = lax.axis_index("s") * NC + lax.axis_index("c")
    nbase = wid * nodes_w
    pltpu.sync_copy(b_hbm, b_v)

    def body(it, carry):
      noff = nbase + it * nch
      ioff = noff * 9
      pltpu.sync_copy(idx_hbm.at[pl.ds(ioff, ich)], idx_v)
      pltpu.async_copy(z_hbm.at[idx_v], rows_v, sem).wait()
      bias_vec = b_v[...]
      lane9 = lax.iota(jnp.int32, 16) * 9

      def group(g, c):
        base = g * 144
        acc = bias_vec
        for d in range(9):
          acc = acc + plsc.load_gather(rows_v, [lane9 + (base + d)])
        out_v[pl.ds(g * 16, 16)] = jnp.maximum(acc, 0.0)
        return c

      lax.fori_loop(0, nch // 16, group, 0)
      pltpu.sync_copy(out_v, out_hbm.at[pl.ds(noff, nch)])
      return carry

    lax.fori_loop(0, n_it, body, 0)

  return k(z, fidx, bias)


def _tc_matmul(a, w, b=None, relu=False, w2=None, block_rows=2048):
  N, K = a.shape
  O = w.shape[1]
  O2 = w2.shape[1] if w2 is not None else O
  grid = (pl.cdiv(N, block_rows),)

  def body(*refs):
    if b is not None and w2 is not None:
      a_ref, w_ref, b_ref, w2_ref, o_ref = refs
    elif b is not None:
      a_ref, w_ref, b_ref, o_ref = refs
      w2_ref = None
    else:
      a_ref, w_ref, o_ref = refs
      b_ref = w2_ref = None
    h = jnp.dot(a_ref[...], w_ref[...], preferred_element_type=jnp.float32)
    if b_ref is not None:
      h = h + b_ref[...]
    if relu:
      h = jnp.maximum(h, 0.0)
    if w2_ref is not None:
      h = jnp.dot(h, w2_ref[...], preferred_element_type=jnp.float32)
    o_ref[...] = h

  in_specs = [pl.BlockSpec((block_rows, K), lambda i: (i, 0)),
              pl.BlockSpec((K, O), lambda i: (0, 0))]
  args = [a, w]
  if b is not None:
    in_specs.append(pl.BlockSpec((1, O), lambda i: (0, 0)))
    args.append(b)
  if w2 is not None:
    in_specs.append(pl.BlockSpec((O, O2), lambda i: (0, 0)))
    args.append(w2)

  return pl.pallas_call(
      body,
      grid=grid,
      in_specs=in_specs,
      out_specs=pl.BlockSpec((block_rows, O2), lambda i: (i, 0)),
      out_shape=jax.ShapeDtypeStruct((N, O2), jnp.float32),
  )(*args)


def kernel(features_3, features_4, features_5, features_6,
           neighbors_3, neighbors_4, neighbors_5, neighbors_6,
           W1, b1, W2, b2, W3, b3, W4, b4):
  sizes = [f.shape[0] for f in
           (features_3, features_4, features_5, features_6)]
  offs = [0, sizes[0], sizes[0] + sizes[1], sizes[0] + sizes[1] + sizes[2]]
  N = sum(sizes)

  x = jnp.concatenate(
      [features_3, features_4, features_5, features_6], axis=0)[:, 0]
  nidx = jnp.concatenate(
      [neighbors_3 + offs[0], neighbors_4 + offs[1],
       neighbors_5 + offs[2], neighbors_6 + offs[3]], axis=0)
  flat_idx = nidx.reshape(-1)
  fidx = (nidx * 9 + jnp.arange(9, dtype=jnp.int32)).reshape(-1)

  w1 = jnp.transpose(W1, (2, 1, 0)).reshape(9 * 1, 16)
  w2 = jnp.transpose(W2, (2, 1, 0)).reshape(9 * 16, 32)
  w3 = jnp.transpose(W3, (1, 2, 0)).reshape(32, 9 * 16)
  w4 = W4.reshape(16, 9)
  bb2 = b2.reshape(1, 32)
  bb3 = jnp.broadcast_to(b3, (16,))
  bb4 = jnp.broadcast_to(b4, (16,))
  w1b = jnp.broadcast_to(w1[:, :, None], (9, 16, 16))
  b1b = jnp.broadcast_to(b1[:, None], (16, 16))
  w4mb = jnp.broadcast_to(w4[:, :, None], (16, 9, 16))
  w4hb = w4mb

  g1 = _sc_gather(x, flat_idx, 6480).reshape(N, 9)
  h1 = _tc_matmul(g1, w1, b1.reshape(1, 16), relu=True)

  g2 = _sc_gather(h1, flat_idx, 3120).reshape(N, 9 * 16)
  z3 = _tc_matmul(g2, w2, bb2, relu=True, w2=w3)

  h3 = _sc_gather9sum16(z3.reshape(N * 9, 16), fidx, bb3, 360)

  z4 = _tc_matmul(h3, w4)
  out = _sc_gather9sum1(z4.reshape(N * 9), fidx, bb4, 720)

  s0, s1, s2, s3 = sizes
  out = out.reshape(N, 1)
  return (out[:s0], out[s0:s0 + s1],
          out[s0 + s1:s0 + s1 + s2], out[s0 + s1 + s2:])
